# baseline (speedup 1.0000x reference)
import functools

import numpy as np
import jax
import jax.numpy as jnp
from jax import lax
from jax.experimental import pallas as pl
from jax.experimental.pallas import tpu as pltpu
from jax.experimental.pallas import tpu_sc as plsc

_EPS = 1e-3


def _rsqrt_precise(s):
    r = lax.rsqrt(s)
    return r * (1.5 - 0.5 * s * r * r)


def _sqrt_precise(s):
    return s * _rsqrt_precise(s)


_INV2PI = np.float32(1.0 / (2.0 * np.pi))
_RC1 = np.float32(6.283203125)
_RC2 = np.float32(-1.781782e-05)
_SINP = tuple(np.float32(v) for v in
              (9.99983615e-01, -1.66630886e-01, 8.31161466e-03,
               -1.93036605e-04, 2.16655721e-06))
_COSP = tuple(np.float32(v) for v in
              (9.99999401e-01, -4.99995301e-01, 4.16607501e-02,
               -1.38617799e-03, 2.42399769e-05, -2.21318664e-07))


def _sincos(h):
    n = jnp.round(h * _INV2PI)
    r = (h - n * _RC1) - n * _RC2
    u = r * r
    s = _SINP
    c = _COSP
    sinr = r * (s[0] + u * (s[1] + u * (s[2] + u * (s[3] + u * s[4]))))
    cosr = c[0] + u * (c[1] + u * (c[2] + u * (c[3] + u * (c[4] + u * c[5]))))
    return sinr, cosr


_SC_NC = 2
_SC_NS = 16
_SC_CH = 2000


def _sc_gather(table, idx, E):
    NW = _SC_NC * _SC_NS
    per_w = E // NW
    n_ch = per_w // _SC_CH
    mesh = plsc.VectorSubcoreMesh(core_axis_name="c", subcore_axis_name="s")

    @functools.partial(
        pl.kernel,
        mesh=mesh,
        out_type=jax.ShapeDtypeStruct((E, 16), jnp.float32),
        compiler_params=pltpu.CompilerParams(use_tc_tiling_on_sc=False),
        scratch_types=[
            pltpu.VMEM((_SC_CH,), jnp.int32),
            pltpu.VMEM((_SC_CH, 16), jnp.float32),
            pltpu.SemaphoreType.DMA,
        ],
    )
    def gk(table_hbm, idx_hbm, out_hbm, idx_v, rows_v, sem):
        wid = lax.axis_index("s") * _SC_NC + lax.axis_index("c")
        base = wid * per_w

        def body(j, carry):
            off = base + j * _SC_CH
            pltpu.sync_copy(idx_hbm.at[pl.ds(off, _SC_CH)], idx_v)
            pltpu.async_copy(table_hbm.at[idx_v], rows_v, sem).wait()
            pltpu.sync_copy(rows_v, out_hbm.at[pl.ds(off, _SC_CH)])
            return carry

        lax.fori_loop(0, n_ch, body, 0)

    return gk(table, idx)


def _frames_body(x_ref, o_ref):
    def g(a):
        return x_ref[a]

    nx, ny, nz = g(0), g(1), g(2)
    cax, cay, caz = g(3), g(4), g(5)
    cx, cy, cz = g(6), g(7), g(8)

    def normed(vx, vy, vz):
        inv = _rsqrt_precise(vx * vx + vy * vy + vz * vz + _EPS)
        return vx * inv, vy * inv, vz * inv

    def cross(a, b):
        ax, ay, az = a
        bx, by, bz = b
        return ay * bz - az * by, az * bx - ax * bz, ax * by - ay * bx

    n1 = normed(nx - cax, ny - cay, nz - caz)
    u2 = normed(cx - cax, cy - cay, cz - caz)
    n2 = normed(*cross(n1, u2))
    n3 = normed(*cross(n1, n2))
    b1 = n1[0] * cax + n1[1] * cay + n1[2] * caz
    b2 = n2[0] * cax + n2[1] * cay + n2[2] * caz
    b3 = n3[0] * cax + n3[1] * cay + n3[2] * caz
    outs = (*n1, *n2, *n3, b1, b2, b3)
    for i in range(12):
        o_ref[i] = outs[i]


def _frames_call(xp):
    nb_ = xp.shape[1]
    return pl.pallas_call(
        _frames_body,
        out_shape=jax.ShapeDtypeStruct((12, nb_, 128), jnp.float32),
    )(xp)


def _dot_raw(a, b):
    return lax.dot_general(
        a, b, (((1,), (0,)), ((), ())),
        preferred_element_type=jnp.float32,
    )


def _split3(x):
    f32 = jnp.float32
    h1 = x.astype(jnp.bfloat16)
    r1 = x - h1.astype(f32)
    h2 = r1.astype(jnp.bfloat16)
    h3 = (r1 - h2.astype(f32)).astype(jnp.bfloat16)
    return h1, h2, h3


def _dot_ds(a, b):
    a1, a2, a3 = _split3(a)
    bh = b.astype(jnp.bfloat16)
    return _dot_raw(a1, bh) + _dot_raw(a2, bh) + _dot_raw(a3, bh)


def _dot_sd(a, b):
    ah = a.astype(jnp.bfloat16)
    b1, b2, b3 = _split3(b)
    return _dot_raw(ah, b1) + _dot_raw(ah, b2) + _dot_raw(ah, b3)


def _dot_ds2(a, b):
    f32 = jnp.float32
    a1 = a.astype(jnp.bfloat16)
    a2 = (a - a1.astype(f32)).astype(jnp.bfloat16)
    bh = b.astype(jnp.bfloat16)
    return _dot_raw(a1, bh) + _dot_raw(a2, bh)


def _main_body(BE, R, K, xj_ref, rd_ref, bc_ref, m1a_ref, m1d_ref, gsum_ref,
               bv_ref, bd_ref, o_ref):
    xj = xj_ref[...]
    rd = rd_ref[...]
    bc = bc_ref[...]
    ed = _dot_sd(bc, rd)
    xi = ed[:, 0:16]
    n_e = ed[:, 16:25]
    xy = jnp.concatenate([xi, xj], axis=1)
    dx = _dot_ds(xy, m1a_ref[...])
    td9 = _dot_ds(xy, m1d_ref[...])
    s = dx * dx
    ss = s[:, 0:64] + s[:, 64:128] + s[:, 128:192]
    D = _sqrt_precise(ss + _EPS)
    f32 = jnp.float32
    tdb = td9.astype(jnp.bfloat16).astype(f32)
    neb = n_e.astype(jnp.bfloat16).astype(f32)
    t_ji = _dot_ds2(neb * tdb, gsum_ref[...])
    hv = _dot_raw(t_ji, bv_ref[...])
    hd = _dot_raw(D, bd_ref[...])
    sv, cv = _sincos(hv)
    sd, cd = _sincos(hd)
    o_ref[...] = jnp.concatenate([cv + cd, sv + sd], axis=1)


def _main_call(Xj, row_data, M1a, M1d, Gsum, Bv, Bd, E, K, BE):
    R = BE // K
    grid = E // BE
    bc = jnp.asarray(
        (np.arange(BE)[:, None] // K == np.arange(R)[None, :])
        .astype(np.float32)).astype(jnp.bfloat16)
    body = functools.partial(_main_body, BE, R, K)
    full = lambda shape: pl.BlockSpec(shape, lambda i: (0, 0))
    return pl.pallas_call(
        body,
        grid=(grid,),
        in_specs=[
            pl.BlockSpec((BE, 16), lambda i: (i, 0)),
            pl.BlockSpec((R, 28), lambda i: (i, 0)),
            full(bc.shape),
            full(M1a.shape),
            full(M1d.shape),
            full(Gsum.shape),
            full(Bv.shape),
            full(Bd.shape),
        ],
        out_specs=pl.BlockSpec((BE, 128), lambda i: (i, 0)),
        out_shape=jax.ShapeDtypeStruct((E, 128), jnp.float32),
    )(Xj, row_data, bc, M1a, M1d, Gsum, Bv, Bd)


def _structural_consts():
    def rc(p, d):
        return 3 * p + d if p < 4 else 16 + 3 * (p - 4) + d

    m1a = np.zeros((32, 192), np.float32)
    for d in range(3):
        for m in range(64):
            a, b = m // 8, m % 8
            m1a[rc(b, d), 64 * d + m] += 1.0
            m1a[rc(a, d), 64 * d + m] -= 1.0
    m1d = np.zeros((32, 9), np.float32)
    for c in range(3):
        for d in range(3):
            m1d[19 + d, 3 * c + d] = 1.0
            m1d[3 + d, 3 * c + d] = -1.0
    gsum = np.zeros((9, 3), np.float32)
    for c in range(3):
        for d in range(3):
            gsum[3 * c + d, c] = 1.0
    return jnp.asarray(m1a), jnp.asarray(m1d), jnp.asarray(gsum)


def kernel(X, edge_idx, C, B_vec, B_dist):
    nb, L, K = edge_idx.shape
    E = nb * L * K
    Xf = X.reshape(nb * L, 12)
    table = jnp.pad(Xf, ((0, 0), (0, 4)))
    idx = edge_idx.reshape(E).astype(jnp.int32)
    Xj = _sc_gather(table, idx, E)

    Lp = ((nb * L + 127) // 128) * 128
    xt = X.reshape(nb * L, 4, 3)[:, :3, :].transpose(1, 2, 0).reshape(9, nb * L)
    xp = jnp.pad(xt, ((0, 0), (0, Lp - nb * L))).reshape(9, Lp // 128, 128)
    fr = _frames_call(xp)
    frT = fr.reshape(12, Lp)[:, : nb * L].T
    row_data = jnp.concatenate(
        [Xf, jnp.zeros((nb * L, 4), jnp.float32), frT], axis=1)

    M1a, M1d, Gsum = _structural_consts()
    out = _main_call(Xj, row_data, M1a, M1d, Gsum, B_vec, B_dist, E, K,
                     BE=6400)
    return out.reshape(nb, L, K, 2 * B_vec.shape[1])


# speedup vs baseline: 1.2836x; 1.2836x over previous
import functools

import numpy as np
import jax
import jax.numpy as jnp
from jax import lax
from jax.experimental import pallas as pl
from jax.experimental.pallas import tpu as pltpu
from jax.experimental.pallas import tpu_sc as plsc

_EPS = 1e-3


def _rsqrt_precise(s):
    r = lax.rsqrt(s)
    return r * (1.5 - 0.5 * s * r * r)


def _sqrt_precise(s):
    return s * _rsqrt_precise(s)


_INV2PI = np.float32(1.0 / (2.0 * np.pi))
_RC1 = np.float32(6.283203125)
_RC2 = np.float32(-1.781782e-05)
_SINP = tuple(np.float32(v) for v in
              (9.99983615e-01, -1.66630886e-01, 8.31161466e-03,
               -1.93036605e-04, 2.16655721e-06))
_COSP = tuple(np.float32(v) for v in
              (9.99999401e-01, -4.99995301e-01, 4.16607501e-02,
               -1.38617799e-03, 2.42399769e-05, -2.21318664e-07))


def _sincos(h):
    n = jnp.round(h * _INV2PI)
    r = (h - n * _RC1) - n * _RC2
    u = r * r
    s = _SINP
    c = _COSP
    sinr = r * (s[0] + u * (s[1] + u * (s[2] + u * (s[3] + u * s[4]))))
    cosr = c[0] + u * (c[1] + u * (c[2] + u * (c[3] + u * (c[4] + u * c[5]))))
    return sinr, cosr


_SC_NC = 2
_SC_NS = 16
_SC_CH = 2000


def _sc_gather(table, idx, E):
    NW = _SC_NC * _SC_NS
    per_w = E // NW
    n_ch = per_w // _SC_CH
    mesh = plsc.VectorSubcoreMesh(core_axis_name="c", subcore_axis_name="s")

    @functools.partial(
        pl.kernel,
        mesh=mesh,
        out_type=jax.ShapeDtypeStruct((E, 16), jnp.float32),
        compiler_params=pltpu.CompilerParams(use_tc_tiling_on_sc=False),
        scratch_types=[
            pltpu.VMEM((_SC_CH,), jnp.int32),
            pltpu.VMEM((_SC_CH, 16), jnp.float32),
            pltpu.SemaphoreType.DMA,
        ],
    )
    def gk(table_hbm, idx_hbm, out_hbm, idx_v, rows_v, sem):
        wid = lax.axis_index("s") * _SC_NC + lax.axis_index("c")
        base = wid * per_w

        def body(j, carry):
            off = base + j * _SC_CH
            pltpu.sync_copy(idx_hbm.at[pl.ds(off, _SC_CH)], idx_v)
            pltpu.async_copy(table_hbm.at[idx_v], rows_v, sem).wait()
            pltpu.sync_copy(rows_v, out_hbm.at[pl.ds(off, _SC_CH)])
            return carry

        lax.fori_loop(0, n_ch, body, 0)

    return gk(table, idx)


def _frames_body(x_ref, o_ref):
    def g(a):
        return x_ref[a]

    nx, ny, nz = g(0), g(1), g(2)
    cax, cay, caz = g(3), g(4), g(5)
    cx, cy, cz = g(6), g(7), g(8)

    def normed(vx, vy, vz):
        inv = _rsqrt_precise(vx * vx + vy * vy + vz * vz + _EPS)
        return vx * inv, vy * inv, vz * inv

    def cross(a, b):
        ax, ay, az = a
        bx, by, bz = b
        return ay * bz - az * by, az * bx - ax * bz, ax * by - ay * bx

    n1 = normed(nx - cax, ny - cay, nz - caz)
    u2 = normed(cx - cax, cy - cay, cz - caz)
    n2 = normed(*cross(n1, u2))
    n3 = normed(*cross(n1, n2))
    outs = tuple(
        v.astype(jnp.bfloat16).astype(jnp.float32) for v in (*n1, *n2, *n3))
    for i in range(9):
        o_ref[i] = outs[i]


def _frames_call(xp):
    nb_ = xp.shape[1]
    return pl.pallas_call(
        _frames_body,
        out_shape=jax.ShapeDtypeStruct((9, nb_, 128), jnp.float32),
    )(xp)


def _dot_raw(a, b):
    return lax.dot_general(
        a, b, (((1,), (0,)), ((), ())),
        preferred_element_type=jnp.float32,
    )


def _split3(x):
    f32 = jnp.float32
    h1 = x.astype(jnp.bfloat16)
    r1 = x - h1.astype(f32)
    h2 = r1.astype(jnp.bfloat16)
    h3 = (r1 - h2.astype(f32)).astype(jnp.bfloat16)
    return h1, h2, h3


def _dot_ds(a, b):
    a1, a2, a3 = _split3(a)
    bh = b.astype(jnp.bfloat16)
    return _dot_raw(a1, bh) + _dot_raw(a2, bh) + _dot_raw(a3, bh)


def _dot_sd(a, b):
    ah = a.astype(jnp.bfloat16)
    b1, b2, b3 = _split3(b)
    return _dot_raw(ah, b1) + _dot_raw(ah, b2) + _dot_raw(ah, b3)


def _dot_ds2(a, b):
    f32 = jnp.float32
    a1 = a.astype(jnp.bfloat16)
    a2 = (a - a1.astype(f32)).astype(jnp.bfloat16)
    bh = b.astype(jnp.bfloat16)
    return _dot_raw(a1, bh) + _dot_raw(a2, bh)


def _main_body(BE, R, K, xj_ref, rd_ref, bc_ref, m1a_ref, gsum_ref,
               bv_ref, bd_ref, o_ref):
    xj = xj_ref[...]
    rd = rd_ref[...]
    bc = bc_ref[...]
    ed = _dot_sd(bc, rd)
    xi = ed[:, 0:16]
    n_e = ed[:, 16:25]
    xy = jnp.concatenate([xi, xj], axis=1)
    dx = _dot_ds(xy, m1a_ref[...])
    s = dx * dx
    ss = s[:, 0:64] + s[:, 64:128] + s[:, 128:192]
    D = _sqrt_precise(ss + _EPS)
    f32 = jnp.float32
    td3 = xj[:, 3:6] - ed[:, 3:6]
    tdb = td3.astype(jnp.bfloat16).astype(f32)
    td9v = jnp.concatenate([tdb, tdb, tdb], axis=1)
    t_ji = _dot_ds2(n_e * td9v, gsum_ref[...])
    hv = _dot_raw(t_ji, bv_ref[...])
    hd = _dot_raw(D, bd_ref[...])
    sv, cv = _sincos(hv)
    sd, cd = _sincos(hd)
    o_ref[...] = jnp.concatenate([cv + cd, sv + sd], axis=1)


def _main_call(Xj, row_data, M1a, Gsum, Bv, Bd, E, K, BE):
    R = BE // K
    grid = E // BE
    bc = jnp.asarray(
        (np.arange(BE)[:, None] // K == np.arange(R)[None, :])
        .astype(np.float32)).astype(jnp.bfloat16)
    body = functools.partial(_main_body, BE, R, K)
    full = lambda shape: pl.BlockSpec(shape, lambda i: (0, 0))
    return pl.pallas_call(
        body,
        grid=(grid,),
        in_specs=[
            pl.BlockSpec((BE, 16), lambda i: (i, 0)),
            pl.BlockSpec((R, 25), lambda i: (i, 0)),
            full(bc.shape),
            full(M1a.shape),
            full(Gsum.shape),
            full(Bv.shape),
            full(Bd.shape),
        ],
        out_specs=pl.BlockSpec((BE, 128), lambda i: (i, 0)),
        out_shape=jax.ShapeDtypeStruct((E, 128), jnp.float32),
    )(Xj, row_data, bc, M1a, Gsum, Bv, Bd)


def _structural_consts():
    def rc(p, d):
        return 3 * p + d if p < 4 else 16 + 3 * (p - 4) + d

    m1a = np.zeros((32, 192), np.float32)
    for d in range(3):
        for m in range(64):
            a, b = m // 8, m % 8
            m1a[rc(b, d), 64 * d + m] += 1.0
            m1a[rc(a, d), 64 * d + m] -= 1.0
    gsum = np.zeros((9, 3), np.float32)
    for c in range(3):
        for d in range(3):
            gsum[3 * c + d, c] = 1.0
    return jnp.asarray(m1a), jnp.asarray(gsum)


def kernel(X, edge_idx, C, B_vec, B_dist):
    nb, L, K = edge_idx.shape
    E = nb * L * K
    Xf = X.reshape(nb * L, 12)
    table = jnp.pad(Xf, ((0, 0), (0, 4)))
    idx = edge_idx.reshape(E).astype(jnp.int32)
    Xj = _sc_gather(table, idx, E)

    Lp = ((nb * L + 127) // 128) * 128
    xt = X.reshape(nb * L, 4, 3)[:, :3, :].transpose(1, 2, 0).reshape(9, nb * L)
    xp = jnp.pad(xt, ((0, 0), (0, Lp - nb * L))).reshape(9, Lp // 128, 128)
    fr = _frames_call(xp)
    frT = fr.reshape(9, Lp)[:, : nb * L].T
    row_data = jnp.concatenate(
        [Xf, jnp.zeros((nb * L, 4), jnp.float32), frT], axis=1)

    M1a, Gsum = _structural_consts()
    out = _main_call(Xj, row_data, M1a, Gsum, B_vec, B_dist, E, K,
                     BE=2560)
    return out.reshape(nb, L, K, 2 * B_vec.shape[1])
